# baseline (speedup 1.0000x reference)
import functools

import jax
import jax.numpy as jnp
from jax import lax
from jax.experimental import pallas as pl
from jax.experimental.pallas import tpu as pltpu
from jax.experimental.pallas import tpu_sc as plsc

N_NODES = 10000
N_EDGES = 320000
F_IN = 128
HIDDEN = 64
N_GRAPHS = 512

NC = 2
NS = 16
NW = NC * NS
CH = 128
NJ = 80
NB = 8
E_PAD = NW * NJ * CH
NJ2 = E_PAD // (NS * CH)
NG2 = NJ2 // NB
N_PAD = 10240
RPT = N_PAD // NS

_MESH = dict(core_axis_name="c", subcore_axis_name="s",
             num_cores=NC, num_subcores=NS)

_HI = lax.Precision.HIGHEST


def _deg_body(row3, degp, rowv, ones_v, zb, acc):
    c = lax.axis_index("c")
    s = lax.axis_index("s")
    wid = s * NC + c
    for i in range(CH // 16):
        ones_v[pl.ds(i * 16, 16)] = jnp.ones((16,), jnp.float32)
    for i in range(RPT // 16):
        zb[pl.ds(i * 16, 16)] = jnp.zeros((16,), jnp.float32)
    pltpu.sync_copy(zb, acc.at[pl.ds(s * RPT, RPT)])
    plsc.subcore_barrier()
    pltpu.sync_copy(row3.at[wid], rowv)

    def body(j, carry):
        pltpu.sync_copy(ones_v, acc.at[rowv.at[j]], add=True)
        return carry

    lax.fori_loop(0, NJ, body, 0)
    plsc.subcore_barrier()
    pltpu.sync_copy(acc.at[pl.ds(s * RPT, RPT)],
                    degp.at[c, pl.ds(s * RPT, RPT)])


@functools.cache
def _deg_kernel():
    return pl.kernel(
        _deg_body,
        out_type=jax.ShapeDtypeStruct((NC, N_PAD), jnp.float32),
        mesh=plsc.VectorSubcoreMesh(**_MESH),
        scratch_types=[
            pltpu.VMEM((NJ, CH), jnp.int32),
            pltpu.VMEM((CH,), jnp.float32),
            pltpu.VMEM((RPT,), jnp.float32),
            pltpu.VMEM_SHARED((N_PAD,), jnp.float32),
        ],
    )


SW = 32


def _make_scat_body(npass):
    def body(zt3, r16, c16, p, rowv, colv, *bufs_and_sems):
        bufs = bufs_and_sems[:NB]
        zbuf, acc = bufs_and_sems[NB], bufs_and_sems[NB + 1]
        sgs = bufs_and_sems[NB + 2:2 * NB + 2]
        sss = bufs_and_sems[2 * NB + 2:]
        c = lax.axis_index("c")
        s = lax.axis_index("s")

        def zbody(i, carry):
            for k in range(SW // 16):
                zbuf[i, pl.ds(k * 16, 16)] = jnp.zeros((16,), jnp.float32)
            return carry

        lax.fori_loop(0, CH, zbody, 0)
        pltpu.sync_copy(r16.at[s], rowv)
        pltpu.sync_copy(c16.at[s], colv)

        def ring(tbl):
            for b in range(NB):
                pltpu.async_copy(tbl.at[colv.at[b]], bufs[b], sgs[b])

            def gbody(g, carry):
                for b in range(NB):
                    j = g * NB + b
                    pltpu.make_async_copy(tbl.at[colv.at[j]], bufs[b],
                                          sgs[b]).wait()
                    pltpu.async_copy(bufs[b], acc.at[rowv.at[j]], sss[b],
                                     add=True)
                for b in range(NB):
                    j = g * NB + b
                    pltpu.make_async_copy(bufs[b], acc.at[rowv.at[j]],
                                          sss[b]).wait()

                    @pl.when(g < NG2 - 1)
                    def _():
                        pltpu.async_copy(tbl.at[colv.at[(g + 1) * NB + b]],
                                         bufs[b], sgs[b])
                return carry

            lax.fori_loop(0, NG2, gbody, 0)

        for t in range(npass):
            for k in range(RPT // CH):
                pltpu.sync_copy(zbuf, acc.at[pl.ds(s * RPT + k * CH, CH)])
            plsc.subcore_barrier()
            ring(zt3.at[c * npass + t])
            plsc.subcore_barrier()
            pltpu.sync_copy(acc.at[pl.ds(s * RPT, RPT)],
                            p.at[c * npass + t, pl.ds(s * RPT, RPT)])

    return body


@functools.cache
def _scat_kernel(npass):
    return pl.kernel(
        _make_scat_body(npass),
        out_type=jax.ShapeDtypeStruct((NC * npass, N_PAD, SW), jnp.float32),
        mesh=plsc.VectorSubcoreMesh(**_MESH),
        scratch_types=(
            [pltpu.VMEM((NJ2, CH), jnp.int32)] * 2
            + [pltpu.VMEM((CH, SW), jnp.float32)] * (NB + 1)
            + [pltpu.VMEM_SHARED((N_PAD, SW), jnp.float32)]
            + [pltpu.SemaphoreType.DMA] * (2 * NB)
        ),
        compiler_params=pltpu.CompilerParams(use_tc_tiling_on_sc=False),
    )


RB = 2000
NRB = N_NODES // RB
FH = F_IN // 2
HH = HIDDEN // 2


def _tc0_body(x_ref, degT_ref, lmax_ref, batchc_ref,
              zt_ref, c_ref, dinv_ref, diag_ref):
    deg = degT_ref[:, 0:1] + degT_ref[:, 1:2]
    safe = jnp.maximum(deg, 1.0)
    dinv = jnp.where(deg > 0, 1.0 / jnp.sqrt(safe), 0.0)
    iota = lax.broadcasted_iota(jnp.int32, (RB, N_GRAPHS), 1)
    oh = (batchc_ref[...] == iota).astype(jnp.float32)
    lam = jnp.dot(oh, lmax_ref[...], preferred_element_type=jnp.float32,
                  precision=_HI)
    ilam = 2.0 / lam
    zt = dinv * x_ref[...]
    for k in range(F_IN // SW):
        zt_ref[k] = zt[:, k * SW:(k + 1) * SW]
    c_ref[...] = ilam * dinv
    dinv_ref[...] = dinv
    diag_ref[...] = ilam - 1.0


_tc0_kernel = pl.pallas_call(
    _tc0_body,
    grid=(NRB,),
    in_specs=[
        pl.BlockSpec((RB, F_IN), lambda i: (i, 0)),
        pl.BlockSpec((RB, NC), lambda i: (i, 0)),
        pl.BlockSpec((N_GRAPHS, 1), lambda i: (0, 0)),
        pl.BlockSpec((RB, 1), lambda i: (i, 0)),
    ],
    out_specs=(
        pl.BlockSpec((F_IN // SW, RB, SW), lambda i: (0, i, 0)),
        pl.BlockSpec((RB, 1), lambda i: (i, 0)),
        pl.BlockSpec((RB, 1), lambda i: (i, 0)),
        pl.BlockSpec((RB, 1), lambda i: (i, 0)),
    ),
    out_shape=(
        jax.ShapeDtypeStruct((F_IN // SW, N_NODES, SW), jnp.float32),
        jax.ShapeDtypeStruct((N_NODES, 1), jnp.float32),
        jax.ShapeDtypeStruct((N_NODES, 1), jnp.float32),
        jax.ShapeDtypeStruct((N_NODES, 1), jnp.float32),
    ),
)


def _tcl_body(z_ref, p_ref, c_ref, dinv_ref, diag_ref, W_ref, b_ref,
              znew_ref, zt_ref):
    z = z_ref[...]
    ptot = jnp.concatenate([p_ref[k] for k in range(p_ref.shape[0])], axis=1)
    tx1 = diag_ref[...] * z - c_ref[...] * ptot
    W = W_ref[...]
    out = (jnp.dot(z, W[0], preferred_element_type=jnp.float32)
           + jnp.dot(tx1, W[1], preferred_element_type=jnp.float32)
           + b_ref[...])
    znew = jnp.maximum(out, 0.0)
    znew_ref[...] = znew
    zt = dinv_ref[...] * znew
    zt_ref[0] = zt[:, :HH]
    zt_ref[1] = zt[:, HH:]


@functools.cache
def _tcl_kernel(fin_w):
    ntbl = fin_w // SW
    return pl.pallas_call(
        _tcl_body,
        grid=(NRB,),
        in_specs=[
            pl.BlockSpec((RB, fin_w), lambda i: (i, 0)),
            pl.BlockSpec((ntbl, RB, SW), lambda i: (0, i, 0)),
            pl.BlockSpec((RB, 1), lambda i: (i, 0)),
            pl.BlockSpec((RB, 1), lambda i: (i, 0)),
            pl.BlockSpec((RB, 1), lambda i: (i, 0)),
            pl.BlockSpec((2, fin_w, HIDDEN), lambda i: (0, 0, 0)),
            pl.BlockSpec((1, HIDDEN), lambda i: (0, 0)),
        ],
        out_specs=(
            pl.BlockSpec((RB, HIDDEN), lambda i: (i, 0)),
            pl.BlockSpec((2, RB, HH), lambda i: (0, i, 0)),
        ),
        out_shape=(
            jax.ShapeDtypeStruct((N_NODES, HIDDEN), jnp.float32),
            jax.ShapeDtypeStruct((2, N_NODES, HH), jnp.float32),
        ),
    )


def _fin_body(z_ref, p_ref, c_ref, diag_ref, batchc_ref, W_ref, b_ref,
              fc1w_ref, fc1b_ref, fc2w_ref, fc2b_ref, out_ref, g_ref):
    i = pl.program_id(0)
    z = z_ref[...]
    ptot = jnp.concatenate([p_ref[0], p_ref[1]], axis=1)
    tx1 = diag_ref[...] * z - c_ref[...] * ptot
    W = W_ref[...]
    out4 = (jnp.dot(z, W[0], preferred_element_type=jnp.float32)
            + jnp.dot(tx1, W[1], preferred_element_type=jnp.float32)
            + b_ref[...])
    h = jnp.maximum(out4, 0.0)
    iota = lax.broadcasted_iota(jnp.int32, (RB, N_GRAPHS), 1)
    oh = (batchc_ref[...] == iota).astype(jnp.float32)
    contrib = lax.dot_general(oh, h, (((0,), (0,)), ((), ())),
                              preferred_element_type=jnp.float32,
                              precision=_HI)

    @pl.when(i == 0)
    def _():
        g_ref[...] = contrib

    @pl.when(i > 0)
    def _():
        g_ref[...] += contrib

    @pl.when(i == NRB - 1)
    def _():
        g = g_ref[...]
        g1 = jnp.maximum(
            jnp.dot(g, fc1w_ref[...], preferred_element_type=jnp.float32)
            + fc1b_ref[...], 0.0)
        out_ref[...] = (
            jnp.dot(g1, fc2w_ref[...], preferred_element_type=jnp.float32)
            + fc2b_ref[...])


_fin_kernel = pl.pallas_call(
    _fin_body,
    grid=(NRB,),
    in_specs=[
        pl.BlockSpec((RB, HIDDEN), lambda i: (i, 0)),
        pl.BlockSpec((NC, RB, HH), lambda i: (0, i, 0)),
        pl.BlockSpec((RB, 1), lambda i: (i, 0)),
        pl.BlockSpec((RB, 1), lambda i: (i, 0)),
        pl.BlockSpec((RB, 1), lambda i: (i, 0)),
        pl.BlockSpec((2, HIDDEN, HIDDEN), lambda i: (0, 0, 0)),
        pl.BlockSpec((1, HIDDEN), lambda i: (0, 0)),
        pl.BlockSpec((HIDDEN, 32), lambda i: (0, 0)),
        pl.BlockSpec((1, 32), lambda i: (0, 0)),
        pl.BlockSpec((32, 1), lambda i: (0, 0)),
        pl.BlockSpec((1, 1), lambda i: (0, 0)),
    ],
    out_specs=pl.BlockSpec((N_GRAPHS, 1), lambda i: (0, 0)),
    out_shape=jax.ShapeDtypeStruct((N_GRAPHS, 1), jnp.float32),
    scratch_shapes=[pltpu.VMEM((N_GRAPHS, HIDDEN), jnp.float32)],
)


def kernel(x, edge_index, lmax, batch,
           W1, b1, W2, b2, W3, b3, W4, b4,
           fc1_w, fc1_b, fc2_w, fc2_b):
    row = edge_index[0]
    col = edge_index[1]
    npad = E_PAD - N_EDGES
    pr = N_NODES + (jnp.arange(npad, dtype=jnp.int32) % (N_PAD - N_NODES))
    pc = jnp.arange(npad, dtype=jnp.int32) % N_NODES
    rw = jnp.concatenate([row, pr])
    cw = jnp.concatenate([col, pc])
    row3 = rw.reshape(NW, NJ, CH)
    r16 = rw.reshape(NS, NJ2, CH)
    c16 = cw.reshape(NS, NJ2, CH)

    degp = _deg_kernel()(row3)
    degT = degp.T
    batchc = batch.reshape(N_NODES, 1)

    zt, cvec, dinv, diag = _tc0_kernel(x, degT, lmax.reshape(N_GRAPHS, 1),
                                       batchc)
    p = _scat_kernel(F_IN // SW // NC)(zt, r16, c16)
    z, zt = _tcl_kernel(F_IN)(x, p, cvec, dinv, diag, W1,
                              b1.reshape(1, HIDDEN))
    for W, b in ((W2, b2), (W3, b3)):
        p = _scat_kernel(1)(zt, r16, c16)
        z, zt = _tcl_kernel(HIDDEN)(z, p, cvec, dinv, diag, W,
                                    b.reshape(1, HIDDEN))
    p = _scat_kernel(1)(zt, r16, c16)
    return _fin_kernel(z, p, cvec, diag, batchc, W4, b4.reshape(1, HIDDEN),
                       fc1_w, fc1_b.reshape(1, 32),
                       fc2_w, fc2_b.reshape(1, 1))
